# baseline (speedup 1.0000x reference)
import functools

import jax
import jax.numpy as jnp
from jax import lax
from jax.experimental import pallas as pl
from jax.experimental.pallas import tpu as pltpu
from jax.experimental.pallas import tpu_sc as plsc

B = 16384
J = 26
V = 1000000
D = 16
NS = 16
NC = 2
BPT = B // NS
DPC = D // NC
V_ALIGNED = 999936
TAIL = V - V_ALIGNED
STAGE = 62592
STAGE_LAST = V_ALIGNED - (NS - 1) * STAGE


def _make_lookup():
    mesh = plsc.VectorSubcoreMesh(core_axis_name="c", subcore_axis_name="s")

    @functools.partial(
        pl.kernel,
        mesh=mesh,
        out_type=jax.ShapeDtypeStruct((J, D, B), jnp.float32),
        scratch_types=[
            pltpu.VMEM_SHARED((V,), jnp.float32),
            pltpu.VMEM_SHARED((V,), jnp.float32),
            pltpu.VMEM((BPT,), jnp.int32),
            pltpu.VMEM((BPT,), jnp.int32),
            pltpu.VMEM((BPT,), jnp.float32),
            pltpu.VMEM((BPT,), jnp.float32),
            pltpu.VMEM((DPC * 128,), jnp.float32),
            pltpu.SemaphoreType.DMA,
            pltpu.SemaphoreType.DMA,
            pltpu.SemaphoreType.DMA,
            pltpu.SemaphoreType.DMA,
            pltpu.SemaphoreType.DMA,
            pltpu.SemaphoreType.DMA,
        ],
    )
    def body(idx_hbm, table_hbm, tail_hbm, out_hbm, row_a, row_b, idx0, idx1,
             vals0, vals1, tail_v, ssem, isem0, isem1, gsem, osem0, osem1):
        c = lax.axis_index("c")
        s = lax.axis_index("s")
        b0 = pl.multiple_of(s * BPT, 128)
        rows = [row_a, row_b]
        idxs = [idx0, idx1]
        isems = [isem0, isem1]
        vals = [vals0, vals1]
        osems = [osem0, osem1]

        tb = pl.multiple_of(c * (DPC * 128), 128)
        pltpu.sync_copy(tail_hbm.at[pl.ds(tb, DPC * 128)], tail_v)

        def stage_descs(d, k):
            off = pl.multiple_of(s * STAGE, 128)
            off_l = (NS - 1) * STAGE
            main = pltpu.make_async_copy(
                table_hbm.at[d, pl.ds(off, STAGE)],
                rows[k].at[pl.ds(off, STAGE)], ssem)
            last = pltpu.make_async_copy(
                table_hbm.at[d, pl.ds(off_l, STAGE_LAST)],
                rows[k].at[pl.ds(off_l, STAGE_LAST)], ssem)
            tail = pltpu.make_async_copy(
                tail_v.at[pl.ds((d % DPC) * 128, TAIL)],
                rows[k].at[pl.ds(V_ALIGNED, TAIL)], ssem)
            return main, last, tail

        def fire_stage(d, k):
            main, last, tail = stage_descs(d, k)

            @pl.when(s < NS - 1)
            def _():
                main.start()

            @pl.when(s == NS - 1)
            def _():
                last.start()
                tail.start()

        def wait_stage(d, k):
            main, last, tail = stage_descs(d, k)

            @pl.when(s < NS - 1)
            def _():
                main.wait()

            @pl.when(s == NS - 1)
            def _():
                last.wait()
                tail.wait()

        def fire_idx(j):
            p = j % 2
            return pltpu.async_copy(idx_hbm.at[j, pl.ds(b0, BPT)],
                                    idxs[p], isems[p])

        def drain_write(p):
            pltpu.make_async_copy(
                table_hbm.at[0, pl.ds(0, BPT)], vals[p], osems[p]).wait()

        for cc in range(NC):

            @pl.when(c == cc)
            def _(cc=cc):
                fire_stage(cc * DPC, 0)
                idx_cps = {0: fire_idx(0), 1: fire_idx(1)}
                writes_fired = [0, 0]
                writes_drained = [0, 0]

                def fire_gather(j, k):
                    p = j % 2
                    return pltpu.async_copy(rows[k].at[idxs[p]], vals[p],
                                            gsem)

                for dd in range(DPC):
                    d = cc * DPC + dd
                    k = dd % 2
                    wait_stage(d, k)
                    plsc.subcore_barrier()
                    if dd < DPC - 1:
                        fire_stage(d + 1, 1 - k)
                    idx_cps[0].wait()
                    if writes_drained[0] < writes_fired[0]:
                        drain_write(0)
                        writes_drained[0] += 1
                    g_cps = {0: fire_gather(0, k)}
                    for j in range(J):
                        p = j % 2
                        q = 1 - p
                        if j + 1 < J:
                            idx_cps[q].wait()
                            if writes_drained[q] < writes_fired[q]:
                                drain_write(q)
                                writes_drained[q] += 1
                            g_cps[q] = fire_gather(j + 1, k)
                        g_cps[p].wait()
                        pltpu.async_copy(
                            vals[p], out_hbm.at[j, d, pl.ds(b0, BPT)],
                            osems[p])
                        writes_fired[p] += 1
                        if j + 2 < J or dd < DPC - 1:
                            idx_cps[p] = fire_idx((j + 2) % J)
                    plsc.subcore_barrier()

                for p in (0, 1):
                    while writes_drained[p] < writes_fired[p]:
                        drain_write(p)
                        writes_drained[p] += 1

    return body


_lookup = _make_lookup()


@jax.jit
def kernel(node_ids, table):
    tail = jnp.pad(table[V_ALIGNED:].T, ((0, 0), (0, 128 - TAIL))).reshape(-1)
    out_t = _lookup(node_ids.T, table.T, tail)
    return jnp.transpose(out_t, (2, 0, 1))


# speedup vs baseline: 1.2913x; 1.2913x over previous
import functools

import jax
import jax.numpy as jnp
from jax import lax
from jax.experimental import pallas as pl
from jax.experimental.pallas import tpu as pltpu
from jax.experimental.pallas import tpu_sc as plsc

B = 16384
J = 26
V = 1000000
D = 16
NS = 16
NC = 2
BPT = B // NS
DPC = D // NC
V_ALIGNED = 999936
TAIL = V - V_ALIGNED
STAGE = 62592
STAGE_LAST = V_ALIGNED - (NS - 1) * STAGE


def _make_lookup():
    mesh = plsc.VectorSubcoreMesh(core_axis_name="c", subcore_axis_name="s")

    @functools.partial(
        pl.kernel,
        mesh=mesh,
        out_type=jax.ShapeDtypeStruct((J, D, B), jnp.float32),
        scratch_types=[
            pltpu.VMEM_SHARED((V,), jnp.float32),
            pltpu.VMEM((J * BPT,), jnp.int32),
            pltpu.VMEM((J * BPT,), jnp.float32),
            pltpu.VMEM((DPC * 128,), jnp.float32),
            pltpu.SemaphoreType.DMA,
            pltpu.SemaphoreType.DMA,
            pltpu.SemaphoreType.DMA,
        ],
    )
    def body(idx_hbm, table_hbm, tail_hbm, out_hbm, row_sh, idx_v, vals_v,
             tail_v, gsem, ssem, osem):
        c = lax.axis_index("c")
        s = lax.axis_index("s")
        b0 = pl.multiple_of(s * BPT, 128)

        for j in range(J):
            pltpu.async_copy(idx_hbm.at[j, pl.ds(b0, BPT)],
                             idx_v.at[pl.ds(j * BPT, BPT)], gsem)
        tb = pl.multiple_of(c * (DPC * 128), 128)
        pltpu.sync_copy(tail_hbm.at[pl.ds(tb, DPC * 128)], tail_v)
        pltpu.make_async_copy(
            table_hbm.at[0, pl.ds(0, J * BPT)], idx_v, gsem).wait()

        def drain_writes():
            pltpu.make_async_copy(
                table_hbm.at[0, pl.ds(0, J * BPT)], vals_v, osem).wait()

        for d in range(D):

            @pl.when(c == d // DPC)
            def _(d=d):
                @pl.when(s < NS - 1)
                def _():
                    off = pl.multiple_of(s * STAGE, 128)
                    pltpu.async_copy(
                        table_hbm.at[d, pl.ds(off, STAGE)],
                        row_sh.at[pl.ds(off, STAGE)], ssem).wait()

                @pl.when(s == NS - 1)
                def _():
                    off = (NS - 1) * STAGE
                    pltpu.async_copy(
                        table_hbm.at[d, pl.ds(off, STAGE_LAST)],
                        row_sh.at[pl.ds(off, STAGE_LAST)], ssem).wait()
                    pltpu.async_copy(
                        tail_v.at[pl.ds((d % DPC) * 128, TAIL)],
                        row_sh.at[pl.ds(V_ALIGNED, TAIL)], ssem).wait()

                plsc.subcore_barrier()
                if d % DPC >= 1:
                    drain_writes()
                pltpu.async_copy(row_sh.at[idx_v], vals_v, gsem).wait()
                plsc.subcore_barrier()
                for j in range(J):
                    pltpu.async_copy(
                        vals_v.at[pl.ds(j * BPT, BPT)],
                        out_hbm.at[j, d, pl.ds(b0, BPT)], osem)

        for cc in range(NC):

            @pl.when(c == cc)
            def _():
                drain_writes()

    return body


_lookup = _make_lookup()


@jax.jit
def kernel(node_ids, table):
    tail = jnp.pad(table[V_ALIGNED:].T, ((0, 0), (0, 128 - TAIL))).reshape(-1)
    out_t = _lookup(node_ids.T, table.T, tail)
    return jnp.transpose(out_t, (2, 0, 1))
